# baseline (speedup 1.0000x reference)
import functools

import jax
import jax.numpy as jnp
from jax import lax
from jax.experimental import pallas as pl
from jax.experimental.pallas import tpu as pltpu
from jax.experimental.pallas import tpu_sc as plsc


def _premul_body(tok_ref, pat_ref, w_ref, t1_ref, p2_ref, t3_ref):
    d = tok_ref.shape[1]
    t = tok_ref[...]
    p = pat_ref[...]
    w = w_ref[...]
    t1_ref[...] = jnp.dot(t, w[0:d, :], preferred_element_type=jnp.float32)
    p2_ref[...] = jnp.dot(p, w[d:2 * d, :], preferred_element_type=jnp.float32)
    t3_ref[...] = jnp.dot(t, w[2 * d:3 * d, :], preferred_element_type=jnp.float32)


def _premultiply(tokens_table, paths_table, W_t):
    n_tok, d = tokens_table.shape
    assert paths_table.shape[0] == n_tok
    rb = 2000
    grid = (n_tok // rb,)
    out_shape = [jax.ShapeDtypeStruct((n_tok, d), jnp.float32)] * 3
    return pl.pallas_call(
        _premul_body,
        grid=grid,
        in_specs=[
            pl.BlockSpec((rb, d), lambda r: (r, 0)),
            pl.BlockSpec((rb, d), lambda r: (r, 0)),
            pl.BlockSpec((3 * d, d), lambda r: (0, 0)),
        ],
        out_specs=[pl.BlockSpec((rb, d), lambda r: (r, 0))] * 3,
        out_shape=out_shape,
    )(tokens_table, paths_table, W_t)


def _gather_sum(T1, P2, T3, i1, i2, i3):
    nr = i1.shape[0]
    d = T1.shape[1]
    h = d // 2
    info = plsc.get_sparse_core_info()
    nc, ns = info.num_cores, info.num_subcores
    nw = nc * ns
    chunk = 80
    per_w = nr // nw
    n_chunks = per_w // chunk
    assert per_w * nw == nr and n_chunks * chunk == per_w and n_chunks % 4 == 0

    @functools.partial(
        pl.kernel,
        mesh=plsc.VectorSubcoreMesh(core_axis_name="c", subcore_axis_name="s"),
        compiler_params=pltpu.CompilerParams(needs_layout_passes=False),
        out_type=jax.ShapeDtypeStruct((nr, h), jnp.uint32),
        scratch_types=[
            pltpu.VMEM((2, 3, chunk), jnp.int32),
            pltpu.VMEM((2, 3, chunk, d), jnp.float32),
            pltpu.VMEM((4, chunk, h), jnp.uint32),
        ] + [pltpu.SemaphoreType.DMA] * 8,
    )
    def sc_kernel(i1_hbm, i2_hbm, i3_hbm, t1_hbm, p2_hbm, t3_hbm, out_hbm,
                  idx_v, rows_v, out_v,
                  isem0, isem1, gsem0, gsem1, osem0, osem1, osem2, osem3):
        isem = (isem0, isem1)
        gsem = (gsem0, gsem1)
        osem = (osem0, osem1, osem2, osem3)
        i_hbm = (i1_hbm, i2_hbm, i3_hbm)
        t_hbm = (t1_hbm, p2_hbm, t3_hbm)
        wid = lax.axis_index("s") * nc + lax.axis_index("c")
        base0 = wid * per_w

        def idx_src(c, j):
            return i_hbm[j].at[pl.ds(base0 + c * chunk, chunk)]

        def out_dst(c):
            return out_hbm.at[pl.ds(base0 + c * chunk, chunk)]

        def fire_gathers(b):
            for j in range(3):
                pltpu.async_copy(t_hbm[j].at[idx_v.at[b, j]], rows_v.at[b, j],
                                 gsem[b])

        def drain_gathers(b):
            for j in range(3):
                pltpu.make_async_copy(t_hbm[j].at[idx_v.at[b, j]],
                                      rows_v.at[b, j], gsem[b]).wait()

        def combine(b, o):
            def row(r, carry):
                for k in range(h // 16):
                    sl_e = pl.ds(k * 16, 16)
                    sl_o = pl.ds(h + k * 16, 16)
                    se = (rows_v[b, 0, r, sl_e] + rows_v[b, 1, r, sl_e]
                          + rows_v[b, 2, r, sl_e])
                    so = (rows_v[b, 0, r, sl_o] + rows_v[b, 1, r, sl_o]
                          + rows_v[b, 2, r, sl_o])
                    packed = plsc.pack(se, so, format=plsc.PackFormat.INTERLEAVED)
                    out_v[o, r, sl_e] = plsc.bitcast(packed, jnp.uint32)
                return carry

            lax.fori_loop(0, chunk, row, 0)

        for c in (0, 1):
            for j in range(3):
                pltpu.sync_copy(idx_src(c, j), idx_v.at[c, j])
            fire_gathers(c)

        def group(g, carry):
            c0 = g * 4
            for u in range(4):
                b = u % 2
                o = u
                c = c0 + u
                drain_gathers(b)

                @pl.when(c + 2 < n_chunks)
                def _fire_idx():
                    for j in range(3):
                        pltpu.async_copy(idx_src(c + 2, j), idx_v.at[b, j],
                                         isem[b])

                @pl.when(c >= 4)
                def _wait_old_out():
                    pltpu.make_async_copy(out_v.at[o], out_dst(c - 4),
                                          osem[o]).wait()

                combine(b, o)
                pltpu.async_copy(out_v.at[o], out_dst(c), osem[o])

                @pl.when(c + 2 < n_chunks)
                def _prefetch():
                    for j in range(3):
                        pltpu.make_async_copy(idx_src(c + 2, j), idx_v.at[b, j],
                                              isem[b]).wait()
                    fire_gathers(b)

            return carry

        lax.fori_loop(0, n_chunks // 4, group, 0)

        for u in range(4):
            c = n_chunks - 4 + u
            pltpu.make_async_copy(out_v.at[u], out_dst(c), osem[u]).wait()

    return sc_kernel(i1, i2, i3, T1, P2, T3)


def _attend_body(s_ref, bte_ref, bto_ref, wae_ref, wao_ref, oute_ref, outo_ref):
    w = s_ref[...]
    f_e = lax.bitcast_convert_type(w << jnp.uint32(16), jnp.float32)
    f_o = lax.bitcast_convert_type(w & jnp.uint32(0xFFFF0000), jnp.float32)
    t_e = jnp.tanh(f_e + bte_ref[...][None, None, :])
    t_o = jnp.tanh(f_o + bto_ref[...][None, None, :])
    logits = jnp.sum(t_e * wae_ref[...][None, None, :]
                     + t_o * wao_ref[...][None, None, :], axis=2)
    m = jnp.max(logits, axis=1, keepdims=True)
    e = jnp.exp(logits - m)
    attn = e / jnp.sum(e, axis=1, keepdims=True)
    oute_ref[...] = jnp.sum(t_e * attn[:, :, None], axis=1)
    outo_ref[...] = jnp.sum(t_o * attn[:, :, None], axis=1)


def _attend(s, bt_e, bt_o, wa_e, wa_o):
    bsz, l, h = s.shape
    bb = 64
    grid = (bsz // bb,)
    vec = pl.BlockSpec((h,), lambda i: (0,))
    return pl.pallas_call(
        _attend_body,
        grid=grid,
        in_specs=[pl.BlockSpec((bb, l, h), lambda i: (i, 0, 0)),
                  vec, vec, vec, vec],
        out_specs=[pl.BlockSpec((bb, h), lambda i: (i, 0))] * 2,
        out_shape=[jax.ShapeDtypeStruct((bsz, h), jnp.float32)] * 2,
    )(s, bt_e, bt_o, wa_e, wa_o)


def kernel(contexts, tokens_table, paths_table, W_t, b_t, W_a, b_a):
    bsz, l = contexts.shape[1], contexts.shape[2]
    d = tokens_table.shape[1]
    h = d // 2
    perm = jnp.concatenate([jnp.arange(0, d, 2), jnp.arange(1, d, 2)])
    T1, P2, T3 = _premultiply(tokens_table, paths_table, W_t[:, perm])
    s = _gather_sum(T1, P2, T3, contexts[0].reshape(-1),
                    contexts[1].reshape(-1), contexts[2].reshape(-1))
    wa = W_a.reshape(-1)
    out_e, out_o = _attend(s.reshape(bsz, l, h),
                           b_t[0::2], b_t[1::2], wa[0::2], wa[1::2])
    return jnp.stack([out_e, out_o], axis=-1).reshape(bsz, d)
